# Initial kernel scaffold; baseline (speedup 1.0000x reference)
import jax
import jax.numpy as jnp
from jax.experimental import pallas as pl


def kernel(x):
    raise NotImplementedError("write your pallas kernel here")


# speedup vs baseline: 1.1747x; 1.1747x over previous
import jax
import jax.numpy as jnp
from jax.experimental import pallas as pl

_N = 33554432
_COLS = 1024
_ROWS = _N // _COLS
_BLOCK_ROWS = 2048
_GRID = _ROWS // _BLOCK_ROWS


def _isin_body(x_ref, o_ref):
    x = x_ref[...]
    m = (x == 0.0) | (x == 2.0) | (x == 4.0) | (x == 6.0) | (x == 8.0)
    o_ref[...] = m


def kernel(x):
    x2 = x.reshape(_ROWS, _COLS)
    out = pl.pallas_call(
        _isin_body,
        grid=(_GRID,),
        in_specs=[pl.BlockSpec((_BLOCK_ROWS, _COLS), lambda i: (i, 0))],
        out_specs=pl.BlockSpec((_BLOCK_ROWS, _COLS), lambda i: (i, 0)),
        out_shape=jax.ShapeDtypeStruct((_ROWS, _COLS), jnp.bool_),
    )(x2)
    return out.reshape(_N)
